# Initial kernel scaffold; baseline (speedup 1.0000x reference)
import jax
import jax.numpy as jnp
from jax.experimental import pallas as pl


def kernel(boxes, classification, translation, rotation):
    raise NotImplementedError("write your pallas kernel here")


# speedup vs baseline: 37.1744x; 37.1744x over previous
import jax
import jax.numpy as jnp
from jax.experimental import pallas as pl

_NUM_CLASSES = 8
_N = 20000
_NP = 20480
_ROWS = _NP // 128
_MAX_DET = 100
_NMS_T = 0.5
_SCORE_T = 0.01
_NEGV = -1e9
_BIG = 2**30


def _nms_class_kernel(sc_ref, x1_ref, y1_ref, x2_ref, y2_ref, oidx_ref, osc_ref):
    X1 = x1_ref[...]
    Y1 = y1_ref[...]
    X2 = x2_ref[...]
    Y2 = y2_ref[...]
    area = (X2 - X1) * (Y2 - Y1)

    row_i = jax.lax.broadcasted_iota(jnp.int32, (_ROWS, 128), 0)
    col_i = jax.lax.broadcasted_iota(jnp.int32, (_ROWS, 128), 1)
    flat = row_i * 128 + col_i
    in_range = flat < _N

    S = sc_ref[0]
    S = jnp.where((S >= _SCORE_T) & in_range, S, _NEGV)

    slot = jax.lax.broadcasted_iota(jnp.int32, (1, 128), 1)
    idxs0 = jnp.full((1, 128), -1, dtype=jnp.int32)
    sels0 = jnp.full((1, 128), _NEGV, dtype=jnp.float32)

    def body(i, carry):
        S, idxs, sels = carry
        m = jnp.max(S)
        best = jnp.min(jnp.where(S == m, flat, _BIG))
        valid = m > _NEGV / 2
        sel_mask = flat == best
        bx1 = jnp.sum(jnp.where(sel_mask, X1, 0.0))
        by1 = jnp.sum(jnp.where(sel_mask, Y1, 0.0))
        bx2 = jnp.sum(jnp.where(sel_mask, X2, 0.0))
        by2 = jnp.sum(jnp.where(sel_mask, Y2, 0.0))
        xx1 = jnp.maximum(bx1, X1)
        yy1 = jnp.maximum(by1, Y1)
        xx2 = jnp.minimum(bx2, X2)
        yy2 = jnp.minimum(by2, Y2)
        inter = jnp.clip(xx2 - xx1, 0.0) * jnp.clip(yy2 - yy1, 0.0)
        barea = (bx2 - bx1) * (by2 - by1)
        iou = inter / (barea + area - inter + 1e-8)
        S = jnp.where(valid & (iou > _NMS_T), _NEGV, S)
        S = jnp.where(sel_mask, _NEGV, S)
        at_i = slot == i
        idxs = jnp.where(at_i, jnp.where(valid, best, -1), idxs)
        sels = jnp.where(at_i, jnp.where(valid, m, _NEGV), sels)
        return S, idxs, sels

    _, idxs, sels = jax.lax.fori_loop(0, _MAX_DET, body, (S, idxs0, sels0))
    oidx_ref[0] = idxs
    osc_ref[0] = sels


def _merge_kernel(sc_ref, idx_ref, x1_ref, y1_ref, x2_ref, y2_ref,
                  t0_ref, t1_ref, t2_ref, r0_ref, r1_ref, r2_ref,
                  of_ref, oi_ref):
    SC = sc_ref[...]
    IDX = idx_ref[...]
    fields = (x1_ref[...], y1_ref[...], x2_ref[...], y2_ref[...],
              t0_ref[...], t1_ref[...], t2_ref[...],
              r0_ref[...], r1_ref[...], r2_ref[...])

    r8 = jax.lax.broadcasted_iota(jnp.int32, (_NUM_CLASSES, 128), 0)
    c8 = jax.lax.broadcasted_iota(jnp.int32, (_NUM_CLASSES, 128), 1)
    flat8 = r8 * 128 + c8

    rowN = jax.lax.broadcasted_iota(jnp.int32, (_ROWS, 128), 0)
    colN = jax.lax.broadcasted_iota(jnp.int32, (_ROWS, 128), 1)
    flatN = rowN * 128 + colN

    r16 = jax.lax.broadcasted_iota(jnp.int32, (16, 128), 0)
    c16 = jax.lax.broadcasted_iota(jnp.int32, (16, 128), 1)
    cI = jax.lax.broadcasted_iota(jnp.int32, (_NUM_CLASSES, 128), 1)

    OF0 = jnp.full((16, 128), -1.0, dtype=jnp.float32)
    OI0 = jnp.full((_NUM_CLASSES, 128), -1, dtype=jnp.int32)

    def body(j, carry):
        SC, OF, OI = carry
        m = jnp.max(SC)
        bf = jnp.min(jnp.where(SC == m, flat8, _BIG))
        valid = m > _NEGV / 2
        sel8 = flat8 == bf
        gidx = jnp.sum(jnp.where(sel8, IDX, 0))
        lab = jnp.right_shift(bf, 7)
        gclip = jnp.maximum(gidx, 0)
        selN = flatN == gclip
        vals = [jnp.sum(jnp.where(selN, f, 0.0)) for f in fields]
        vals.append(m)
        out_col = jnp.full((16, 128), -1.0, dtype=jnp.float32)
        for r, v in enumerate(vals):
            out_col = jnp.where(r16 == r, jnp.where(valid, v, -1.0), out_col)
        OF = jnp.where(c16 == j, out_col, OF)
        OI = jnp.where(cI == j, jnp.where(valid, lab, -1), OI)
        SC = jnp.where(sel8, _NEGV, SC)
        return SC, OF, OI

    _, OF, OI = jax.lax.fori_loop(0, _MAX_DET, body, (SC, OF0, OI0))
    of_ref[...] = OF
    oi_ref[...] = OI


def _pad_col(v):
    return jnp.pad(v, (0, _NP - _N)).reshape(_ROWS, 128)


def kernel(boxes, classification, translation, rotation):
    b = boxes[0]
    cls = classification[0]
    t = translation[0]
    r = rotation[0]

    x1 = _pad_col(b[:, 0])
    y1 = _pad_col(b[:, 1])
    x2 = _pad_col(b[:, 2])
    y2 = _pad_col(b[:, 3])
    scores = jnp.pad(cls.T, ((0, 0), (0, _NP - _N)), constant_values=-1.0)
    scores = scores.reshape(_NUM_CLASSES, _ROWS, 128)

    cls_idx, cls_sc = pl.pallas_call(
        _nms_class_kernel,
        grid=(_NUM_CLASSES,),
        in_specs=[
            pl.BlockSpec((1, _ROWS, 128), lambda c: (c, 0, 0)),
            pl.BlockSpec((_ROWS, 128), lambda c: (0, 0)),
            pl.BlockSpec((_ROWS, 128), lambda c: (0, 0)),
            pl.BlockSpec((_ROWS, 128), lambda c: (0, 0)),
            pl.BlockSpec((_ROWS, 128), lambda c: (0, 0)),
        ],
        out_specs=[
            pl.BlockSpec((1, 1, 128), lambda c: (c, 0, 0)),
            pl.BlockSpec((1, 1, 128), lambda c: (c, 0, 0)),
        ],
        out_shape=[
            jax.ShapeDtypeStruct((_NUM_CLASSES, 1, 128), jnp.int32),
            jax.ShapeDtypeStruct((_NUM_CLASSES, 1, 128), jnp.float32),
        ],
    )(scores, x1, y1, x2, y2)

    cls_idx = cls_idx.reshape(_NUM_CLASSES, 128)
    cls_sc = cls_sc.reshape(_NUM_CLASSES, 128)

    t0 = _pad_col(t[:, 0])
    t1 = _pad_col(t[:, 1])
    t2 = _pad_col(t[:, 2])
    r0 = _pad_col(r[:, 0])
    r1 = _pad_col(r[:, 1])
    r2 = _pad_col(r[:, 2])

    OF, OI = pl.pallas_call(
        _merge_kernel,
        out_shape=[
            jax.ShapeDtypeStruct((16, 128), jnp.float32),
            jax.ShapeDtypeStruct((_NUM_CLASSES, 128), jnp.int32),
        ],
    )(cls_sc, cls_idx, x1, y1, x2, y2, t0, t1, t2, r0, r1, r2)

    out_b = OF[0:4, :_MAX_DET].T
    out_t = OF[4:7, :_MAX_DET].T
    out_r = OF[7:10, :_MAX_DET].T
    out_s = OF[10, :_MAX_DET]
    out_l = OI[0, :_MAX_DET]
    return (out_b[None], out_s[None], out_l[None], out_t[None], out_r[None])


# speedup vs baseline: 149.2411x; 4.0146x over previous
import functools

import jax
import jax.numpy as jnp
from jax import lax
from jax.experimental import pallas as pl
from jax.experimental.pallas import tpu as pltpu
from jax.experimental.pallas import tpu_sc as plsc

_NC = 8
_N = 20000
_NP = 20480
_NV = _NP // 16
_NL1V = _NV // 16
_NL2V = _NL1V // 16
_MD = 100
_SLOTS = 112
_NMS_T = 0.5
_SCORE_T = 0.01
_NEG = -1e9
_BIG = 2**30
_SENT = 2e9


def _lane():
    return lax.iota(jnp.int32, 16)


def _splat_f(x):
    return jnp.full((16,), x, dtype=jnp.float32)


def _splat_i(x):
    return jnp.full((16,), x, dtype=jnp.int32)


def _sc_body(scores_h, x1_h, y1_h, x2_h, y2_h, ftab_h,
             of_h, os_h, ol_h,
             sc_v, x1_v, y1_v, x2_v, y2_v, l1_v, l2_v,
             ax1_v, ay1_v, ax2_v, ay2_v, osc_v, oidx_v,
             sh_sc, sh_idx, msc_v, midx_v, sout_v, lout_v, gidx_v, rows_v,
             dsem):
    core = lax.axis_index("c")
    sub = lax.axis_index("s")
    lane = _lane()
    active = (core == 0) & (sub < _NC)

    @pl.when(active)
    def _per_class():
        pltpu.sync_copy(scores_h.at[pl.ds(sub * _NP, _NP)], sc_v)
        pltpu.sync_copy(x1_h, x1_v)
        pltpu.sync_copy(y1_h, y1_v)
        pltpu.sync_copy(x2_h, x2_v)
        pltpu.sync_copy(y2_h, y2_v)

        def build_l1(j, _):
            def inner(k, acc):
                i = j * 16 + k
                v = sc_v[pl.ds(i * 16, 16)]
                v = jnp.where(v >= _SCORE_T, v, _NEG)
                sc_v[pl.ds(i * 16, 16)] = v
                return jnp.where(lane == k, jnp.max(v), acc)
            l1_v[pl.ds(j * 16, 16)] = lax.fori_loop(0, 16, inner, _splat_f(_NEG))
            return 0

        lax.fori_loop(0, _NL1V, build_l1, 0)

        def build_l2(j, _):
            def inner(k, acc):
                m = jnp.max(l1_v[pl.ds((j * 16 + k) * 16, 16)])
                return jnp.where(lane == k, m, acc)
            l2_v[pl.ds(j * 16, 16)] = lax.fori_loop(0, 16, inner, _splat_f(_NEG))
            return 0

        lax.fori_loop(0, _NL2V, build_l2, 0)

        def init_slots(t, _):
            ax1_v[pl.ds(t * 16, 16)] = _splat_f(_SENT)
            ay1_v[pl.ds(t * 16, 16)] = _splat_f(_SENT)
            ax2_v[pl.ds(t * 16, 16)] = _splat_f(_SENT)
            ay2_v[pl.ds(t * 16, 16)] = _splat_f(_SENT)
            osc_v[pl.ds(t * 16, 16)] = _splat_f(_NEG)
            oidx_v[pl.ds(t * 16, 16)] = _splat_i(-1)
            return 0

        lax.fori_loop(0, _SLOTS // 16, init_slots, 0)

        def greedy_cond(state):
            count, done = state
            return (count < _MD) & jnp.logical_not(done)

        def greedy_body(state):
            count, done = state
            m2 = _splat_f(_NEG)
            for t in range(_NL2V):
                m2 = jnp.maximum(m2, l2_v[pl.ds(t * 16, 16)])
            s = jnp.max(m2)
            valid = s > _NEG / 2
            g1 = _BIG
            for t in range(_NL2V):
                v2 = l2_v[pl.ds(t * 16, 16)]
                g1 = jnp.minimum(
                    g1, jnp.min(jnp.where(v2 == s, lane + t * 16, _BIG)))
            g1 = jnp.minimum(g1, _NL1V - 1)
            v1 = l1_v[pl.ds(g1 * 16, 16)]
            p1 = jnp.min(jnp.where(v1 == s, lane, _BIG))
            p1 = jnp.minimum(p1, 15)
            g0 = g1 * 16 + p1
            v0 = sc_v[pl.ds(g0 * 16, 16)]
            p0 = jnp.min(jnp.where(v0 == s, lane, _BIG))
            p0 = jnp.minimum(p0, 15)
            gi = g0 * 16 + p0
            v0n = jnp.where(lane == p0, _NEG, v0)
            sc_v[pl.ds(g0 * 16, 16)] = v0n
            v1n = jnp.where(lane == p1, jnp.max(v0n), v1)
            l1_v[pl.ds(g1 * 16, 16)] = v1n
            g2 = g1 // 16
            p2 = g1 - g2 * 16
            v2 = l2_v[pl.ds(g2 * 16, 16)]
            l2_v[pl.ds(g2 * 16, 16)] = jnp.where(lane == p2, jnp.max(v1n), v2)
            gis = _splat_i(0) + gi
            bx1 = plsc.load_gather(x1_v, [gis])
            by1 = plsc.load_gather(y1_v, [gis])
            bx2 = plsc.load_gather(x2_v, [gis])
            by2 = plsc.load_gather(y2_v, [gis])
            barea = (bx2 - bx1) * (by2 - by1)

            def chk(t, anyov):
                qx1 = ax1_v[pl.ds(t * 16, 16)]
                qy1 = ay1_v[pl.ds(t * 16, 16)]
                qx2 = ax2_v[pl.ds(t * 16, 16)]
                qy2 = ay2_v[pl.ds(t * 16, 16)]
                xx1 = jnp.maximum(qx1, bx1)
                yy1 = jnp.maximum(qy1, by1)
                xx2 = jnp.minimum(qx2, bx2)
                yy2 = jnp.minimum(qy2, by2)
                inter = (jnp.maximum(xx2 - xx1, 0.0)
                         * jnp.maximum(yy2 - yy1, 0.0))
                qarea = (qx2 - qx1) * (qy2 - qy1)
                iou = inter / (qarea + barea - inter + 1e-8)
                return anyov | (iou > _NMS_T)

            anyov = lax.fori_loop(0, _SLOTS // 16, chk,
                                  jnp.zeros((16,), dtype=jnp.bool_))
            accept = valid & jnp.logical_not(jnp.any(anyov))
            base = (count // 16) * 16
            wm = (lane == (count - base)) & accept
            ax1_v[pl.ds(base, 16)] = jnp.where(wm, bx1, ax1_v[pl.ds(base, 16)])
            ay1_v[pl.ds(base, 16)] = jnp.where(wm, by1, ay1_v[pl.ds(base, 16)])
            ax2_v[pl.ds(base, 16)] = jnp.where(wm, bx2, ax2_v[pl.ds(base, 16)])
            ay2_v[pl.ds(base, 16)] = jnp.where(wm, by2, ay2_v[pl.ds(base, 16)])
            osc_v[pl.ds(base, 16)] = jnp.where(
                wm, _splat_f(0.0) + s, osc_v[pl.ds(base, 16)])
            oidx_v[pl.ds(base, 16)] = jnp.where(
                wm, gis, oidx_v[pl.ds(base, 16)])
            count = count + jnp.where(accept, 1, 0)
            return count, jnp.logical_not(valid)

        lax.while_loop(greedy_cond, greedy_body, (jnp.int32(0), jnp.bool_(False)))

        pltpu.sync_copy(osc_v, sh_sc.at[pl.ds(sub * _SLOTS, _SLOTS)])
        pltpu.sync_copy(oidx_v, sh_idx.at[pl.ds(sub * _SLOTS, _SLOTS)])

    plsc.subcore_barrier()

    @pl.when((core == 0) & (sub == 0))
    def _merge():
        pltpu.sync_copy(sh_sc, msc_v)
        pltpu.sync_copy(sh_idx, midx_v)
        cbase = jnp.where(lane < _NC, lane * _SLOTS, 0)

        def init_out(t, _):
            sout_v[pl.ds(t * 16, 16)] = _splat_f(-1.0)
            lout_v[pl.ds(t * 16, 16)] = _splat_i(-1)
            gidx_v[pl.ds(t * 16, 16)] = _splat_i(_N)
            return 0

        lax.fori_loop(0, _SLOTS // 16, init_out, 0)

        def merge_step(j, p):
            addr = cbase + jnp.minimum(p, _SLOTS - 1)
            h = plsc.load_gather(msc_v, [addr])
            h = jnp.where(lane < _NC, h, _NEG)
            m = jnp.max(h)
            valid = m > _NEG / 2
            bl = jnp.min(jnp.where(h == m, lane, _BIG))
            bl = jnp.minimum(bl, _NC - 1)
            gidx16 = plsc.load_gather(midx_v, [addr])
            gi = jnp.max(jnp.where(lane == bl, gidx16, -1))
            base = (j // 16) * 16
            wm = lane == (j - base)
            sout_v[pl.ds(base, 16)] = jnp.where(
                wm, jnp.where(valid, _splat_f(0.0) + m, -1.0),
                sout_v[pl.ds(base, 16)])
            lout_v[pl.ds(base, 16)] = jnp.where(
                wm, jnp.where(valid, _splat_i(0) + bl, -1),
                lout_v[pl.ds(base, 16)])
            gidx_v[pl.ds(base, 16)] = jnp.where(
                wm, jnp.where(valid, jnp.maximum(_splat_i(0) + gi, 0), _N),
                gidx_v[pl.ds(base, 16)])
            return p + jnp.where((lane == bl) & valid, 1, 0)

        lax.fori_loop(0, _MD, merge_step, _splat_i(0))

        pltpu.async_copy(ftab_h.at[gidx_v], rows_v, dsem).wait()
        pltpu.sync_copy(rows_v, of_h)
        pltpu.sync_copy(sout_v, os_h)
        pltpu.sync_copy(lout_v, ol_h)


_mesh = plsc.VectorSubcoreMesh(core_axis_name="c", subcore_axis_name="s")

_sc_call = functools.partial(
    pl.kernel,
    mesh=_mesh,
    compiler_params=pltpu.CompilerParams(needs_layout_passes=False),
    out_type=[
        jax.ShapeDtypeStruct((_SLOTS, 128), jnp.float32),
        jax.ShapeDtypeStruct((_SLOTS,), jnp.float32),
        jax.ShapeDtypeStruct((_SLOTS,), jnp.int32),
    ],
    scratch_types=[
        pltpu.VMEM((_NP,), jnp.float32),
        pltpu.VMEM((_NP,), jnp.float32),
        pltpu.VMEM((_NP,), jnp.float32),
        pltpu.VMEM((_NP,), jnp.float32),
        pltpu.VMEM((_NP,), jnp.float32),
        pltpu.VMEM((_NV,), jnp.float32),
        pltpu.VMEM((_NL1V,), jnp.float32),
        pltpu.VMEM((_SLOTS,), jnp.float32),
        pltpu.VMEM((_SLOTS,), jnp.float32),
        pltpu.VMEM((_SLOTS,), jnp.float32),
        pltpu.VMEM((_SLOTS,), jnp.float32),
        pltpu.VMEM((_SLOTS,), jnp.float32),
        pltpu.VMEM((_SLOTS,), jnp.int32),
        pltpu.VMEM_SHARED((_NC * _SLOTS,), jnp.float32),
        pltpu.VMEM_SHARED((_NC * _SLOTS,), jnp.int32),
        pltpu.VMEM((_NC * _SLOTS,), jnp.float32),
        pltpu.VMEM((_NC * _SLOTS,), jnp.int32),
        pltpu.VMEM((_SLOTS,), jnp.float32),
        pltpu.VMEM((_SLOTS,), jnp.int32),
        pltpu.VMEM((_SLOTS,), jnp.int32),
        pltpu.VMEM((_SLOTS, 128), jnp.float32),
        pltpu.SemaphoreType.DMA,
    ],
)(_sc_body)


def kernel(boxes, classification, translation, rotation):
    b = boxes[0]
    c = classification[0]
    t = translation[0]
    r = rotation[0]

    scores = jnp.pad(c.T, ((0, 0), (0, _NP - _N)), constant_values=-1.0)
    scores = scores.reshape(-1)
    x1 = jnp.pad(b[:, 0], (0, _NP - _N))
    y1 = jnp.pad(b[:, 1], (0, _NP - _N))
    x2 = jnp.pad(b[:, 2], (0, _NP - _N))
    y2 = jnp.pad(b[:, 3], (0, _NP - _N))
    ftab = jnp.concatenate(
        [b, t, r, jnp.zeros((_N, 118), jnp.float32)], axis=1)
    ftab = jnp.pad(ftab, ((0, 8), (0, 0)), constant_values=-1.0)

    of, os, ol = _sc_call(scores, x1, y1, x2, y2, ftab)

    out_b = of[:_MD, 0:4]
    out_t = of[:_MD, 4:7]
    out_r = of[:_MD, 7:10]
    out_s = os[:_MD]
    out_l = ol[:_MD]
    return (out_b[None], out_s[None], out_l[None], out_t[None], out_r[None])


# speedup vs baseline: 262.3916x; 1.7582x over previous
import functools

import jax
import jax.numpy as jnp
from jax import lax
from jax.experimental import pallas as pl
from jax.experimental.pallas import tpu as pltpu
from jax.experimental.pallas import tpu_sc as plsc

_NC = 8
_N = 20000
_NP = 20480
_NV = _NP // 16
_NL1V = _NV // 16
_NL2V = _NL1V // 16
_MD = 100
_SLOTS = 112
_NMS_T = 0.5
_SCORE_T = 0.01
_NEG = -1e9
_BIG = 2**30
_SENT = 2e9


def _lane():
    return lax.iota(jnp.int32, 16)


def _splat_f(x):
    return jnp.full((16,), x, dtype=jnp.float32)


def _splat_i(x):
    return jnp.full((16,), x, dtype=jnp.int32)


def _sc_body(scores_h, x1_h, y1_h, x2_h, y2_h,
             t0_h, t1_h, t2_h, r0_h, r1_h, r2_h,
             of_h, os_h, ol_h,
             sc_v, x1_v, y1_v, x2_v, y2_v, l1_v, l2_v,
             ax1_v, ay1_v, ax2_v, ay2_v, osc_v, oidx_v,
             sh_sc, sh_idx, msc_v, midx_v, sout_v, lout_v, gidx_v, big_v,
             dsem):
    core = lax.axis_index("c")
    sub = lax.axis_index("s")
    lane = _lane()
    active = (core == 0) & (sub < _NC)

    @pl.when(active)
    def _per_class():
        pltpu.sync_copy(scores_h.at[pl.ds(sub * _NP, _NP)], sc_v)
        pltpu.sync_copy(x1_h, x1_v)
        pltpu.sync_copy(y1_h, y1_v)
        pltpu.sync_copy(x2_h, x2_v)
        pltpu.sync_copy(y2_h, y2_v)

        def build_l1(j, _):
            def inner(k, acc):
                i = j * 16 + k
                v = sc_v[pl.ds(i * 16, 16)]
                v = jnp.where(v >= _SCORE_T, v, _NEG)
                sc_v[pl.ds(i * 16, 16)] = v
                return jnp.where(lane == k, jnp.max(v), acc)
            l1_v[pl.ds(j * 16, 16)] = lax.fori_loop(0, 16, inner, _splat_f(_NEG))
            return 0

        lax.fori_loop(0, _NL1V, build_l1, 0)

        def build_l2(j, _):
            def inner(k, acc):
                m = jnp.max(l1_v[pl.ds((j * 16 + k) * 16, 16)])
                return jnp.where(lane == k, m, acc)
            l2_v[pl.ds(j * 16, 16)] = lax.fori_loop(0, 16, inner, _splat_f(_NEG))
            return 0

        lax.fori_loop(0, _NL2V, build_l2, 0)

        def init_slots(t, _):
            ax1_v[pl.ds(t * 16, 16)] = _splat_f(_SENT)
            ay1_v[pl.ds(t * 16, 16)] = _splat_f(_SENT)
            ax2_v[pl.ds(t * 16, 16)] = _splat_f(_SENT)
            ay2_v[pl.ds(t * 16, 16)] = _splat_f(_SENT)
            osc_v[pl.ds(t * 16, 16)] = _splat_f(_NEG)
            oidx_v[pl.ds(t * 16, 16)] = _splat_i(-1)
            return 0

        lax.fori_loop(0, _SLOTS // 16, init_slots, 0)

        def greedy_cond(state):
            count, done = state
            return (count < _MD) & jnp.logical_not(done)

        def greedy_body(state):
            count, done = state
            m2 = _splat_f(_NEG)
            for t in range(_NL2V):
                m2 = jnp.maximum(m2, l2_v[pl.ds(t * 16, 16)])
            s = jnp.max(m2)
            valid = s > _NEG / 2
            g1 = _BIG
            for t in range(_NL2V):
                v2 = l2_v[pl.ds(t * 16, 16)]
                g1 = jnp.minimum(
                    g1, jnp.min(jnp.where(v2 == s, lane + t * 16, _BIG)))
            g1 = jnp.minimum(g1, _NL1V - 1)
            v1 = l1_v[pl.ds(g1 * 16, 16)]
            p1 = jnp.min(jnp.where(v1 == s, lane, _BIG))
            p1 = jnp.minimum(p1, 15)
            g0 = g1 * 16 + p1
            v0 = sc_v[pl.ds(g0 * 16, 16)]
            p0 = jnp.min(jnp.where(v0 == s, lane, _BIG))
            p0 = jnp.minimum(p0, 15)
            gi = g0 * 16 + p0
            v0n = jnp.where(lane == p0, _NEG, v0)
            sc_v[pl.ds(g0 * 16, 16)] = v0n
            v1n = jnp.where(lane == p1, jnp.max(v0n), v1)
            l1_v[pl.ds(g1 * 16, 16)] = v1n
            g2 = g1 // 16
            p2 = g1 - g2 * 16
            v2 = l2_v[pl.ds(g2 * 16, 16)]
            l2_v[pl.ds(g2 * 16, 16)] = jnp.where(lane == p2, jnp.max(v1n), v2)
            gis = _splat_i(0) + gi
            bx1 = plsc.load_gather(x1_v, [gis])
            by1 = plsc.load_gather(y1_v, [gis])
            bx2 = plsc.load_gather(x2_v, [gis])
            by2 = plsc.load_gather(y2_v, [gis])
            barea = (bx2 - bx1) * (by2 - by1)

            def chk(t, anyov):
                qx1 = ax1_v[pl.ds(t * 16, 16)]
                qy1 = ay1_v[pl.ds(t * 16, 16)]
                qx2 = ax2_v[pl.ds(t * 16, 16)]
                qy2 = ay2_v[pl.ds(t * 16, 16)]
                xx1 = jnp.maximum(qx1, bx1)
                yy1 = jnp.maximum(qy1, by1)
                xx2 = jnp.minimum(qx2, bx2)
                yy2 = jnp.minimum(qy2, by2)
                inter = (jnp.maximum(xx2 - xx1, 0.0)
                         * jnp.maximum(yy2 - yy1, 0.0))
                qarea = (qx2 - qx1) * (qy2 - qy1)
                iou = inter / (qarea + barea - inter + 1e-8)
                return anyov | (iou > _NMS_T)

            anyov = lax.fori_loop(0, _SLOTS // 16, chk,
                                  jnp.zeros((16,), dtype=jnp.bool_))
            accept = valid & jnp.logical_not(jnp.any(anyov))
            base = (count // 16) * 16
            wm = (lane == (count - base)) & accept
            ax1_v[pl.ds(base, 16)] = jnp.where(wm, bx1, ax1_v[pl.ds(base, 16)])
            ay1_v[pl.ds(base, 16)] = jnp.where(wm, by1, ay1_v[pl.ds(base, 16)])
            ax2_v[pl.ds(base, 16)] = jnp.where(wm, bx2, ax2_v[pl.ds(base, 16)])
            ay2_v[pl.ds(base, 16)] = jnp.where(wm, by2, ay2_v[pl.ds(base, 16)])
            osc_v[pl.ds(base, 16)] = jnp.where(
                wm, _splat_f(0.0) + s, osc_v[pl.ds(base, 16)])
            oidx_v[pl.ds(base, 16)] = jnp.where(
                wm, gis, oidx_v[pl.ds(base, 16)])
            count = count + jnp.where(accept, 1, 0)
            return count, jnp.logical_not(valid)

        lax.while_loop(greedy_cond, greedy_body, (jnp.int32(0), jnp.bool_(False)))

        pltpu.sync_copy(osc_v, sh_sc.at[pl.ds(sub * _SLOTS, _SLOTS)])
        pltpu.sync_copy(oidx_v, sh_idx.at[pl.ds(sub * _SLOTS, _SLOTS)])

    plsc.subcore_barrier()

    @pl.when((core == 0) & (sub == 0))
    def _merge():
        pltpu.sync_copy(sh_sc, msc_v)
        pltpu.sync_copy(sh_idx, midx_v)
        cbase = jnp.where(lane < _NC, lane * _SLOTS, 0)

        def init_out(t, _):
            sout_v[pl.ds(t * 16, 16)] = _splat_f(-1.0)
            lout_v[pl.ds(t * 16, 16)] = _splat_i(-1)
            gidx_v[pl.ds(t * 16, 16)] = _splat_i(0)
            return 0

        lax.fori_loop(0, _SLOTS // 16, init_out, 0)

        def merge_step(j, p):
            addr = cbase + jnp.minimum(p, _SLOTS - 1)
            h = plsc.load_gather(msc_v, [addr])
            h = jnp.where(lane < _NC, h, _NEG)
            m = jnp.max(h)
            valid = m > _NEG / 2
            bl = jnp.min(jnp.where(h == m, lane, _BIG))
            bl = jnp.minimum(bl, _NC - 1)
            gidx16 = plsc.load_gather(midx_v, [addr])
            gi = jnp.max(jnp.where(lane == bl, gidx16, -1))
            base = (j // 16) * 16
            wm = lane == (j - base)
            sout_v[pl.ds(base, 16)] = jnp.where(
                wm, jnp.where(valid, _splat_f(0.0) + m, -1.0),
                sout_v[pl.ds(base, 16)])
            lout_v[pl.ds(base, 16)] = jnp.where(
                wm, jnp.where(valid, _splat_i(0) + bl, -1),
                lout_v[pl.ds(base, 16)])
            gidx_v[pl.ds(base, 16)] = jnp.where(
                wm, jnp.where(valid, jnp.maximum(_splat_i(0) + gi, 0), 0),
                gidx_v[pl.ds(base, 16)])
            return p + jnp.where((lane == bl) & valid, 1, 0)

        lax.fori_loop(0, _MD, merge_step, _splat_i(0))

        srcs = (x1_h, y1_h, x2_h, y2_h, t0_h, t1_h, t2_h, r0_h, r1_h, r2_h)
        copies = [
            pltpu.async_copy(src.at[gidx_v],
                             big_v.at[pl.ds(f * _SLOTS, _SLOTS)], dsem)
            for f, src in enumerate(srcs)
        ]
        for cp in copies:
            cp.wait()

        def mask_fields(t, _):
            vmask = sout_v[pl.ds(t * 16, 16)] >= 0.0
            for f in range(10):
                o = f * _SLOTS + t * 16
                big_v[pl.ds(o, 16)] = jnp.where(
                    vmask, big_v[pl.ds(o, 16)], -1.0)
            return 0

        lax.fori_loop(0, _SLOTS // 16, mask_fields, 0)
        pltpu.sync_copy(big_v, of_h)
        pltpu.sync_copy(sout_v, os_h)
        pltpu.sync_copy(lout_v, ol_h)


_mesh = plsc.VectorSubcoreMesh(core_axis_name="c", subcore_axis_name="s")

_sc_call = functools.partial(
    pl.kernel,
    mesh=_mesh,
    compiler_params=pltpu.CompilerParams(needs_layout_passes=False),
    out_type=[
        jax.ShapeDtypeStruct((10 * _SLOTS,), jnp.float32),
        jax.ShapeDtypeStruct((_SLOTS,), jnp.float32),
        jax.ShapeDtypeStruct((_SLOTS,), jnp.int32),
    ],
    scratch_types=[
        pltpu.VMEM((_NP,), jnp.float32),
        pltpu.VMEM((_NP,), jnp.float32),
        pltpu.VMEM((_NP,), jnp.float32),
        pltpu.VMEM((_NP,), jnp.float32),
        pltpu.VMEM((_NP,), jnp.float32),
        pltpu.VMEM((_NV,), jnp.float32),
        pltpu.VMEM((_NL1V,), jnp.float32),
        pltpu.VMEM((_SLOTS,), jnp.float32),
        pltpu.VMEM((_SLOTS,), jnp.float32),
        pltpu.VMEM((_SLOTS,), jnp.float32),
        pltpu.VMEM((_SLOTS,), jnp.float32),
        pltpu.VMEM((_SLOTS,), jnp.float32),
        pltpu.VMEM((_SLOTS,), jnp.int32),
        pltpu.VMEM_SHARED((_NC * _SLOTS,), jnp.float32),
        pltpu.VMEM_SHARED((_NC * _SLOTS,), jnp.int32),
        pltpu.VMEM((_NC * _SLOTS,), jnp.float32),
        pltpu.VMEM((_NC * _SLOTS,), jnp.int32),
        pltpu.VMEM((_SLOTS,), jnp.float32),
        pltpu.VMEM((_SLOTS,), jnp.int32),
        pltpu.VMEM((_SLOTS,), jnp.int32),
        pltpu.VMEM((10 * _SLOTS,), jnp.float32),
        pltpu.SemaphoreType.DMA,
    ],
)(_sc_body)


def kernel(boxes, classification, translation, rotation):
    b = boxes[0]
    c = classification[0]
    t = translation[0]
    r = rotation[0]

    scores = jnp.pad(c.T, ((0, 0), (0, _NP - _N)), constant_values=-1.0)
    scores = scores.reshape(-1)
    x1 = jnp.pad(b[:, 0], (0, _NP - _N))
    y1 = jnp.pad(b[:, 1], (0, _NP - _N))
    x2 = jnp.pad(b[:, 2], (0, _NP - _N))
    y2 = jnp.pad(b[:, 3], (0, _NP - _N))
    t0 = jnp.pad(t[:, 0], (0, _NP - _N))
    t1 = jnp.pad(t[:, 1], (0, _NP - _N))
    t2 = jnp.pad(t[:, 2], (0, _NP - _N))
    r0 = jnp.pad(r[:, 0], (0, _NP - _N))
    r1 = jnp.pad(r[:, 1], (0, _NP - _N))
    r2 = jnp.pad(r[:, 2], (0, _NP - _N))

    of, os, ol = _sc_call(scores, x1, y1, x2, y2, t0, t1, t2, r0, r1, r2)

    m = of.reshape(10, _SLOTS)
    out_b = m[0:4, :_MD].T
    out_t = m[4:7, :_MD].T
    out_r = m[7:10, :_MD].T
    out_s = os[:_MD]
    out_l = ol[:_MD]
    return (out_b[None], out_s[None], out_l[None], out_t[None], out_r[None])
